# baseline (speedup 1.0000x reference)
import jax
import jax.numpy as jnp
from jax.experimental import pallas as pl
from jax.experimental.pallas import tpu as pltpu

_CONF = 0.25
_IOU = 0.45
_MAXDET = 1000
_MAXWH = 7680.0
_N = 20000
_NPAD = 20480
_B = 8
_NCLS = 80
_OUTP = 1024
_NBLK = _NPAD // 128
_BMPAD = 256
_NEGINF = float("-inf")
_BIG = 2**30


def _nms_kernel(coords_ref, obj_ref, cls_ref, o_ref,
                m_ref, jf_ref, msk_ref, f_ref, k_ref, bm_ref):
    i = pl.program_id(0)

    @pl.when(i == 0)
    def _init():
        m_ref[...] = jnp.full((_B, _NPAD), _NEGINF, jnp.float32)
        jf_ref[...] = jnp.zeros((_B, _NPAD), jnp.float32)

    @pl.when(i < _NCLS)
    def _class_step():
        prod = cls_ref[0] * obj_ref[...]
        m = m_ref[...]
        upd = prod > m
        cf = i.astype(jnp.float32)
        jf_ref[...] = jnp.where(upd, cf, jf_ref[...])
        m_ref[...] = jnp.where(upd, prod, m)

    @pl.when(i == _NCLS)
    def _greedy():
        obj = obj_ref[...]
        m = m_ref[...]
        jf = jf_ref[...]
        valid = (obj > _CONF) & (m > _CONF)
        msk_ref[...] = jnp.where(valid, m, _NEGINF)

        xc = coords_ref[0]
        yc = coords_ref[1]
        wv = coords_ref[2]
        hv = coords_ref[3]
        off = jf * _MAXWH
        f_ref[0] = (xc - wv / 2.0) + off
        f_ref[1] = (yc - hv / 2.0) + off
        f_ref[2] = (xc + wv / 2.0) + off
        f_ref[3] = (yc + hv / 2.0) + off
        f_ref[4] = jf

        l128 = jax.lax.broadcasted_iota(jnp.int32, (_B, 128), 1)
        l256 = jax.lax.broadcasted_iota(jnp.int32, (_B, _BMPAD), 1)
        olane = jax.lax.broadcasted_iota(jnp.int32, (_B, _OUTP), 1)

        def bm_init(k, bm):
            start = pl.multiple_of(k * 128, 128)
            blk = msk_ref[:, pl.ds(start, 128)]
            bmax = jnp.max(blk, axis=1, keepdims=True)
            return jnp.where(l256 == k, bmax, bm)

        bm0 = jax.lax.fori_loop(
            0, _NBLK, bm_init, jnp.full((_B, _BMPAD), _NEGINF, jnp.float32))
        bm_ref[...] = bm0

        for k in range(6):
            k_ref[k] = jnp.zeros((_B, _OUTP), jnp.float32)

        def pop_next(kcnt):
            bm = bm_ref[...]
            mx = jnp.max(bm, axis=1, keepdims=True)
            has = mx > _NEGINF
            pick = has & (kcnt < _MAXDET)
            bidx = jnp.min(jnp.where(bm == mx, l256, _BIG),
                           axis=1, keepdims=True)
            bidx = jnp.where(has, bidx, 0)

            starts = []
            blks = []
            for im in range(_B):
                st = pl.multiple_of(bidx[im, 0] * 128, 128)
                starts.append(st)
                blks.append(msk_ref[pl.ds(im, 1), pl.ds(st, 128)])
            mblk = jnp.concatenate(blks, axis=0)
            lidx = jnp.min(jnp.where(mblk == mx, l128, _BIG),
                           axis=1, keepdims=True)
            sel = l128 == lidx

            newblk = jnp.where(sel, _NEGINF, mblk)
            for im in range(_B):
                msk_ref[pl.ds(im, 1), pl.ds(starts[im], 128)] = \
                    newblk[im:im + 1, :]
            nbm = jnp.max(newblk, axis=1, keepdims=True)
            bm_ref[...] = jnp.where(l256 == bidx, nbm, bm)

            fblks = []
            for im in range(_B):
                fblks.append(f_ref[:, pl.ds(im, 1), pl.ds(starts[im], 128)])
            fb = jnp.concatenate(fblks, axis=1)
            cf = jnp.sum(jnp.where(sel[None], fb, 0.0),
                         axis=2, keepdims=True)
            return (cf[0], cf[1], cf[2], cf[3], cf[4], mx,
                    pick.astype(jnp.int32))

        def body(carry):
            _, kcnt, cx1, cy1, cx2, cy2, cjf, cmx, cact_i = carry
            cact = cact_i > 0
            carea = (cx2 - cx1) * (cy2 - cy1)

            kx1 = k_ref[0]
            ky1 = k_ref[1]
            kx2 = k_ref[2]
            ky2 = k_ref[3]
            karea = (kx2 - kx1) * (ky2 - ky1)
            inlist = olane < kcnt
            xx1 = jnp.maximum(kx1, cx1)
            yy1 = jnp.maximum(ky1, cy1)
            xx2 = jnp.minimum(kx2, cx2)
            yy2 = jnp.minimum(ky2, cy2)
            inter = jnp.maximum(0.0, xx2 - xx1) * jnp.maximum(0.0, yy2 - yy1)
            iou = inter / (karea + carea - inter + 1e-9)
            sup = jnp.any(inlist & (iou > _IOU), axis=1, keepdims=True)
            keep = cact & ~sup & (kcnt < _MAXDET)

            slot = (olane == kcnt) & keep
            k_ref[0] = jnp.where(slot, cx1, kx1)
            k_ref[1] = jnp.where(slot, cy1, ky1)
            k_ref[2] = jnp.where(slot, cx2, kx2)
            k_ref[3] = jnp.where(slot, cy2, ky2)
            k_ref[4] = jnp.where(slot, cmx, k_ref[4])
            k_ref[5] = jnp.where(slot, cjf, k_ref[5])
            kcnt2 = kcnt + keep.astype(jnp.int32)

            nxt = pop_next(kcnt)
            return (jnp.any(nxt[6] > 0), kcnt2) + nxt

        first = pop_next(jnp.zeros((_B, 1), jnp.int32))
        init = (jnp.any(first[6] > 0),
                jnp.zeros((_B, 1), jnp.int32)) + first
        jax.lax.while_loop(lambda c: c[0], body, init)

        kjf = k_ref[5]
        koff = kjf * _MAXWH
        o_ref[0] = k_ref[0] - koff
        o_ref[1] = k_ref[1] - koff
        o_ref[2] = k_ref[2] - koff
        o_ref[3] = k_ref[3] - koff
        o_ref[4] = k_ref[4]
        o_ref[5] = kjf


def kernel(x):
    pred = x[0]
    pt = jnp.transpose(pred, (2, 0, 1))
    pt = jnp.pad(pt, ((0, 0), (0, 0), (0, _NPAD - _N)))
    coords = pt[0:4]
    obj = pt[4]
    cls = pt[5:5 + _NCLS]

    o = pl.pallas_call(
        _nms_kernel,
        grid=(_NCLS + 1,),
        in_specs=[
            pl.BlockSpec((4, _B, _NPAD), lambda i: (0, 0, 0)),
            pl.BlockSpec((_B, _NPAD), lambda i: (0, 0)),
            pl.BlockSpec((1, _B, _NPAD),
                         lambda i: (jnp.minimum(i, _NCLS - 1), 0, 0)),
        ],
        out_specs=pl.BlockSpec((6, _B, _OUTP), lambda i: (0, 0, 0)),
        out_shape=jax.ShapeDtypeStruct((6, _B, _OUTP), jnp.float32),
        scratch_shapes=[
            pltpu.VMEM((_B, _NPAD), jnp.float32),
            pltpu.VMEM((_B, _NPAD), jnp.float32),
            pltpu.VMEM((_B, _NPAD), jnp.float32),
            pltpu.VMEM((5, _B, _NPAD), jnp.float32),
            pltpu.VMEM((6, _B, _OUTP), jnp.float32),
            pltpu.VMEM((_B, _BMPAD), jnp.float32),
        ],
        compiler_params=pltpu.CompilerParams(
            dimension_semantics=("arbitrary",)),
    )(coords, obj, cls)

    det = jnp.transpose(o, (1, 2, 0))[:, :_MAXDET, :]
    return det


# speedup vs baseline: 1.0066x; 1.0066x over previous
import jax
import jax.numpy as jnp
from jax.experimental import pallas as pl
from jax.experimental.pallas import tpu as pltpu

_CONF = 0.25
_IOU = 0.45
_MAXDET = 1000
_MAXWH = 7680.0
_N = 20000
_NPAD = 20480
_B = 8
_NCLS = 80
_OUTP = 1024
_NBLK = _NPAD // 128
_BMPAD = 256
_NEGINF = float("-inf")
_BIG = 2**30


def _nms_kernel(coords_ref, obj_ref, cls_ref, o_ref,
                m_ref, jf_ref, msk_ref, f_ref, k_ref, bm_ref):
    i = pl.program_id(0)

    @pl.when(i == 0)
    def _init():
        m_ref[...] = jnp.full((_B, _NPAD), _NEGINF, jnp.float32)
        jf_ref[...] = jnp.zeros((_B, _NPAD), jnp.float32)

    @pl.when(i < _NCLS)
    def _class_step():
        prod = cls_ref[0] * obj_ref[...]
        m = m_ref[...]
        upd = prod > m
        cf = i.astype(jnp.float32)
        jf_ref[...] = jnp.where(upd, cf, jf_ref[...])
        m_ref[...] = jnp.where(upd, prod, m)

    @pl.when(i == _NCLS)
    def _greedy():
        obj = obj_ref[...]
        m = m_ref[...]
        jf = jf_ref[...]
        valid = (obj > _CONF) & (m > _CONF)
        msk_ref[...] = jnp.where(valid, m, _NEGINF)

        xc = coords_ref[0]
        yc = coords_ref[1]
        wv = coords_ref[2]
        hv = coords_ref[3]
        off = jf * _MAXWH
        f_ref[0] = (xc - wv / 2.0) + off
        f_ref[1] = (yc - hv / 2.0) + off
        f_ref[2] = (xc + wv / 2.0) + off
        f_ref[3] = (yc + hv / 2.0) + off
        f_ref[4] = jf

        l128 = jax.lax.broadcasted_iota(jnp.int32, (_B, 128), 1)
        l256 = jax.lax.broadcasted_iota(jnp.int32, (_B, _BMPAD), 1)
        olane = jax.lax.broadcasted_iota(jnp.int32, (_B, _OUTP), 1)

        def bm_init(k, bm):
            start = pl.multiple_of(k * 128, 128)
            blk = msk_ref[:, pl.ds(start, 128)]
            bmax = jnp.max(blk, axis=1, keepdims=True)
            return jnp.where(l256 == k, bmax, bm)

        bm0 = jax.lax.fori_loop(
            0, _NBLK, bm_init, jnp.full((_B, _BMPAD), _NEGINF, jnp.float32))
        bm_ref[...] = bm0

        for k in range(6):
            k_ref[k] = jnp.zeros((_B, _OUTP), jnp.float32)

        def pop_next(kcnt):
            bm = bm_ref[...]
            mx = jnp.max(bm, axis=1, keepdims=True)
            has = mx > _NEGINF
            pick = has & (kcnt < _MAXDET)
            bidx = jnp.min(jnp.where(bm == mx, l256, _BIG),
                           axis=1, keepdims=True)
            bidx = jnp.where(has, bidx, 0)

            starts = []
            blks = []
            for im in range(_B):
                st = pl.multiple_of(bidx[im, 0] * 128, 128)
                starts.append(st)
                blks.append(msk_ref[pl.ds(im, 1), pl.ds(st, 128)])
            mblk = jnp.concatenate(blks, axis=0)
            lidx = jnp.min(jnp.where(mblk == mx, l128, _BIG),
                           axis=1, keepdims=True)
            sel = l128 == lidx

            newblk = jnp.where(sel, _NEGINF, mblk)
            for im in range(_B):
                msk_ref[pl.ds(im, 1), pl.ds(starts[im], 128)] = \
                    newblk[im:im + 1, :]
            nbm = jnp.max(newblk, axis=1, keepdims=True)
            bm_ref[...] = jnp.where(l256 == bidx, nbm, bm)

            fblks = []
            for im in range(_B):
                fblks.append(f_ref[:, pl.ds(im, 1), pl.ds(starts[im], 128)])
            fb = jnp.concatenate(fblks, axis=1)
            cf = jnp.sum(jnp.where(sel[None], fb, 0.0),
                         axis=2, keepdims=True)
            return (cf[0], cf[1], cf[2], cf[3], cf[4], mx,
                    pick.astype(jnp.int32))

        def check_and_pop(kcnt, cand):
            cx1, cy1, cx2, cy2, cjf, cmx, cact_i = cand
            cact = cact_i > 0
            carea = (cx2 - cx1) * (cy2 - cy1)

            kx1 = k_ref[0]
            ky1 = k_ref[1]
            kx2 = k_ref[2]
            ky2 = k_ref[3]
            karea = (kx2 - kx1) * (ky2 - ky1)
            inlist = olane < kcnt
            xx1 = jnp.maximum(kx1, cx1)
            yy1 = jnp.maximum(ky1, cy1)
            xx2 = jnp.minimum(kx2, cx2)
            yy2 = jnp.minimum(ky2, cy2)
            inter = jnp.maximum(0.0, xx2 - xx1) * jnp.maximum(0.0, yy2 - yy1)
            iou = inter / (karea + carea - inter + 1e-9)
            sup = jnp.any(inlist & (iou > _IOU), axis=1, keepdims=True)
            keep = cact & ~sup & (kcnt < _MAXDET)

            slot = (olane == kcnt) & keep
            k_ref[0] = jnp.where(slot, cx1, kx1)
            k_ref[1] = jnp.where(slot, cy1, ky1)
            k_ref[2] = jnp.where(slot, cx2, kx2)
            k_ref[3] = jnp.where(slot, cy2, ky2)
            k_ref[4] = jnp.where(slot, cmx, k_ref[4])
            k_ref[5] = jnp.where(slot, cjf, k_ref[5])
            kcnt2 = kcnt + keep.astype(jnp.int32)
            return kcnt2, pop_next(kcnt)

        def body(carry):
            kcnt, cand = carry[1], carry[2:]
            kcnt, cand = check_and_pop(kcnt, cand)
            kcnt, cand = check_and_pop(kcnt, cand)
            return (jnp.any(cand[6] > 0), kcnt) + cand

        first = pop_next(jnp.zeros((_B, 1), jnp.int32))
        init = (jnp.any(first[6] > 0),
                jnp.zeros((_B, 1), jnp.int32)) + first
        jax.lax.while_loop(lambda c: c[0], body, init)

        kjf = k_ref[5]
        koff = kjf * _MAXWH
        o_ref[0] = k_ref[0] - koff
        o_ref[1] = k_ref[1] - koff
        o_ref[2] = k_ref[2] - koff
        o_ref[3] = k_ref[3] - koff
        o_ref[4] = k_ref[4]
        o_ref[5] = kjf


def kernel(x):
    pred = x[0]
    pt = jnp.transpose(pred, (2, 0, 1))
    pt = jnp.pad(pt, ((0, 0), (0, 0), (0, _NPAD - _N)))
    coords = pt[0:4]
    obj = pt[4]
    cls = pt[5:5 + _NCLS]

    o = pl.pallas_call(
        _nms_kernel,
        grid=(_NCLS + 1,),
        in_specs=[
            pl.BlockSpec((4, _B, _NPAD), lambda i: (0, 0, 0)),
            pl.BlockSpec((_B, _NPAD), lambda i: (0, 0)),
            pl.BlockSpec((1, _B, _NPAD),
                         lambda i: (jnp.minimum(i, _NCLS - 1), 0, 0)),
        ],
        out_specs=pl.BlockSpec((6, _B, _OUTP), lambda i: (0, 0, 0)),
        out_shape=jax.ShapeDtypeStruct((6, _B, _OUTP), jnp.float32),
        scratch_shapes=[
            pltpu.VMEM((_B, _NPAD), jnp.float32),
            pltpu.VMEM((_B, _NPAD), jnp.float32),
            pltpu.VMEM((_B, _NPAD), jnp.float32),
            pltpu.VMEM((5, _B, _NPAD), jnp.float32),
            pltpu.VMEM((6, _B, _OUTP), jnp.float32),
            pltpu.VMEM((_B, _BMPAD), jnp.float32),
        ],
        compiler_params=pltpu.CompilerParams(
            dimension_semantics=("arbitrary",)),
    )(coords, obj, cls)

    det = jnp.transpose(o, (1, 2, 0))[:, :_MAXDET, :]
    return det
